# baseline (speedup 1.0000x reference)
import functools

import jax
import jax.numpy as jnp
from jax import lax
from jax.experimental import pallas as pl
from jax.experimental.pallas import tpu as pltpu
from jax.experimental.pallas import tpu_sc as plsc

B, NQ, NK, C, K = 2, 2048, 8192, 256, 16
TQ = 128
TN = 2048


def _topk_body(qt_ref, kc_ref, idx_ref):
    b = pl.program_id(0)
    q = qt_ref[0]
    kc = kc_ref[0]
    qb = q.astype(jnp.bfloat16).astype(jnp.float32)
    kb = kc.astype(jnp.bfloat16).astype(jnp.float32)
    q2 = jnp.zeros((TQ, 1), jnp.float32)
    k2 = jnp.zeros((1, NK), jnp.float32)
    dot = jnp.zeros((TQ, NK), jnp.float32)
    for d in range(3):
        q2 = q2 + q[:, d:d + 1] * q[:, d:d + 1]
        k2 = k2 + kc[d:d + 1, :] * kc[d:d + 1, :]
        dot = dot + qb[:, d:d + 1] * kb[d:d + 1, :]
    dist = (q2 + k2) - 2.0 * dot
    colf = lax.broadcasted_iota(jnp.int32, (TQ, NK), 1).astype(jnp.float32)
    m_prev = jnp.full((TQ, 1), -jnp.inf, jnp.float32)
    a_prev = jnp.full((TQ, 1), -1.0, jnp.float32)
    cols = []
    for _ in range(K):
        keep = (dist > m_prev) | ((dist == m_prev) & (colf > a_prev))
        m = jnp.min(jnp.where(keep, dist, jnp.inf), axis=1, keepdims=True)
        sel = (dist == m) & ((m > m_prev) | (colf > a_prev))
        a = jnp.min(jnp.where(sel, colf, float(NK)), axis=1, keepdims=True)
        cols.append(a)
        m_prev, a_prev = m, a
    idx = jnp.concatenate(cols, axis=1).astype(jnp.int32)
    idx_ref[0] = idx + b * NK


def _topk(query_coords, key_coords):
    qt = jnp.transpose(query_coords, (0, 2, 1))
    return pl.pallas_call(
        _topk_body,
        grid=(B, NQ // TQ),
        in_specs=[
            pl.BlockSpec((1, TQ, 3), lambda b, i: (b, i, 0)),
            pl.BlockSpec((1, 3, NK), lambda b, i: (b, 0, 0)),
        ],
        out_specs=pl.BlockSpec((1, TQ, K), lambda b, i: (b, i, 0)),
        out_shape=jax.ShapeDtypeStruct((B, NQ, K), jnp.int32),
    )(qt, key_coords)


def _kproj_body(x_ref, w_ref, o_ref):
    o_ref[0] = lax.dot_general(x_ref[0], w_ref[...],
                               (((0,), (1,)), ((), ())),
                               preferred_element_type=jnp.float32)


def _qproj_body(x_ref, w_ref, b_ref, o_ref):
    o_ref[0] = lax.dot_general(x_ref[0], w_ref[...],
                               (((0,), (1,)), ((), ())),
                               preferred_element_type=jnp.float32) + b_ref[...]


def _kproj(key_features, w1):
    return pl.pallas_call(
        _kproj_body,
        grid=(B, NK // TN),
        in_specs=[
            pl.BlockSpec((1, C, TN), lambda b, i: (b, 0, i)),
            pl.BlockSpec((C, C), lambda b, i: (0, 0)),
        ],
        out_specs=pl.BlockSpec((1, TN, C), lambda b, i: (b, i, 0)),
        out_shape=jax.ShapeDtypeStruct((B, NK, C), jnp.float32),
    )(key_features, w1)


def _qproj(query_features, w2m1, bias):
    return pl.pallas_call(
        _qproj_body,
        grid=(B, NQ // TN),
        in_specs=[
            pl.BlockSpec((1, C, TN), lambda b, i: (b, 0, i)),
            pl.BlockSpec((C, C), lambda b, i: (0, 0)),
            pl.BlockSpec((1, C), lambda b, i: (0, 0)),
        ],
        out_specs=pl.BlockSpec((1, TN, C), lambda b, i: (b, i, 0)),
        out_shape=jax.ShapeDtypeStruct((B, NQ, C), jnp.float32),
    )(query_features, w2m1, bias)


_NW = 32
_QPW = (B * NQ) // _NW
_QCH = 8
_NCH = _QPW // _QCH
_ROWS = _QCH * K
_CV = C // 16


def _gather_body(table_hbm, idx_hbm, qproj_hbm, out_hbm,
                 idx_v, rows_v, q_v, out_v, sem):
    wid = lax.axis_index("s") * 2 + lax.axis_index("c")

    def chunk(ci, carry):
        qbase = wid * _QPW + ci * _QCH
        ibase = qbase * K
        pltpu.sync_copy(idx_hbm.at[pl.ds(ibase, _ROWS)], idx_v)
        pltpu.sync_copy(qproj_hbm.at[pl.ds(qbase, _QCH)], q_v)
        pltpu.async_copy(table_hbm.at[idx_v], rows_v, sem).wait()

        def per_q(qi, c2):
            def per_c(cj, c3):
                qvec = q_v[qi, pl.ds(cj * 16, 16)]
                acc = jnp.full((16,), -jnp.inf, jnp.float32)
                for k in range(K):
                    x = rows_v[qi * K + k, pl.ds(cj * 16, 16)] + qvec
                    acc = jnp.maximum(acc, jnp.maximum(x, 0.2 * x))
                out_v[qi, pl.ds(cj * 16, 16)] = acc
                return c3
            return lax.fori_loop(0, _CV, per_c, c2)

        lax.fori_loop(0, _QCH, per_q, 0)
        pltpu.sync_copy(out_v, out_hbm.at[pl.ds(qbase, _QCH)])
        return carry

    lax.fori_loop(0, _NCH, chunk, 0)


def _gather_maxpool(table, idx_flat, qproj_flat):
    mesh = plsc.VectorSubcoreMesh(core_axis_name="c", subcore_axis_name="s",
                                  num_cores=2, num_subcores=16)
    fn = functools.partial(
        pl.kernel,
        out_type=jax.ShapeDtypeStruct((B * NQ, C), jnp.float32),
        mesh=mesh,
        scratch_types=[
            pltpu.VMEM((_ROWS,), jnp.int32),
            pltpu.VMEM((_ROWS, C), jnp.float32),
            pltpu.VMEM((_QCH, C), jnp.float32),
            pltpu.VMEM((_QCH, C), jnp.float32),
            pltpu.SemaphoreType.DMA,
        ],
    )(_gather_body)
    return fn(table, idx_flat, qproj_flat)


def kernel(query_coords, query_features, key_coords, key_features, W, b):
    w1 = W[:, :C]
    w2m1 = W[:, C:] - w1
    idx = _topk(query_coords, key_coords)
    kf_proj = _kproj(key_features, w1)
    q_proj = _qproj(query_features, w2m1, b.reshape(1, C))
    out = _gather_maxpool(kf_proj.reshape(B * NK, C),
                          idx.reshape(B * NQ * K),
                          q_proj.reshape(B * NQ, C))
    return out.reshape(B, NQ, C)


# speedup vs baseline: 2.5267x; 2.5267x over previous
import functools

import jax
import jax.numpy as jnp
from jax import lax
from jax.experimental import pallas as pl
from jax.experimental.pallas import tpu as pltpu
from jax.experimental.pallas import tpu_sc as plsc

B, NQ, NK, C, K = 2, 2048, 8192, 256, 16
TQ = 128
TN = 2048


def _topk_body(qt_ref, kc_ref, idx_ref):
    b = pl.program_id(0)
    q = qt_ref[0]
    kc = kc_ref[0]
    qb = q.astype(jnp.bfloat16).astype(jnp.float32)
    kb = kc.astype(jnp.bfloat16).astype(jnp.float32)
    q2 = jnp.zeros((TQ, 1), jnp.float32)
    k2 = jnp.zeros((1, NK), jnp.float32)
    dot = jnp.zeros((TQ, NK), jnp.float32)
    for d in range(3):
        q2 = q2 + q[:, d:d + 1] * q[:, d:d + 1]
        k2 = k2 + kc[d:d + 1, :] * kc[d:d + 1, :]
        dot = dot + qb[:, d:d + 1] * kb[d:d + 1, :]
    dist = (q2 + k2) - 2.0 * dot
    colrev = (NK - lax.broadcasted_iota(jnp.int32, (TQ, NK), 1)).astype(
        jnp.float32)
    cols = []
    for t in range(K):
        m = jnp.min(dist, axis=1, keepdims=True)
        ar = jnp.max(jnp.where(dist == m, colrev, 0.0), axis=1, keepdims=True)
        cols.append(ar)
        if t < K - 1:
            dist = jnp.where(colrev == ar, jnp.inf, dist)
    idx = (float(NK) - jnp.concatenate(cols, axis=1)).astype(jnp.int32)
    idx_ref[0] = idx + b * NK


def _topk(query_coords, key_coords):
    qt = jnp.transpose(query_coords, (0, 2, 1))
    return pl.pallas_call(
        _topk_body,
        grid=(B, NQ // TQ),
        in_specs=[
            pl.BlockSpec((1, TQ, 3), lambda b, i: (b, i, 0)),
            pl.BlockSpec((1, 3, NK), lambda b, i: (b, 0, 0)),
        ],
        out_specs=pl.BlockSpec((1, TQ, K), lambda b, i: (b, i, 0)),
        out_shape=jax.ShapeDtypeStruct((B, NQ, K), jnp.int32),
    )(qt, key_coords)


def _kproj_body(x_ref, w_ref, o_ref):
    o_ref[0] = lax.dot_general(x_ref[0], w_ref[...],
                               (((0,), (1,)), ((), ())),
                               preferred_element_type=jnp.float32)


def _qproj_body(x_ref, w_ref, b_ref, o_ref):
    o_ref[0] = lax.dot_general(x_ref[0], w_ref[...],
                               (((0,), (1,)), ((), ())),
                               preferred_element_type=jnp.float32) + b_ref[...]


def _kproj(key_features, w1):
    return pl.pallas_call(
        _kproj_body,
        grid=(B, NK // TN),
        in_specs=[
            pl.BlockSpec((1, C, TN), lambda b, i: (b, 0, i)),
            pl.BlockSpec((C, C), lambda b, i: (0, 0)),
        ],
        out_specs=pl.BlockSpec((1, TN, C), lambda b, i: (b, i, 0)),
        out_shape=jax.ShapeDtypeStruct((B, NK, C), jnp.float32),
    )(key_features, w1)


def _qproj(query_features, w2m1, bias):
    return pl.pallas_call(
        _qproj_body,
        grid=(B, NQ // TN),
        in_specs=[
            pl.BlockSpec((1, C, TN), lambda b, i: (b, 0, i)),
            pl.BlockSpec((C, C), lambda b, i: (0, 0)),
            pl.BlockSpec((1, C), lambda b, i: (0, 0)),
        ],
        out_specs=pl.BlockSpec((1, TN, C), lambda b, i: (b, i, 0)),
        out_shape=jax.ShapeDtypeStruct((B, NQ, C), jnp.float32),
    )(query_features, w2m1, bias)


_NW = 32
_QPW = (B * NQ) // _NW
_QCH = 8
_NCH = _QPW // _QCH
_ROWS = _QCH * K
_CV = C // 16


def _gather_body(table_hbm, idx_hbm, qproj_hbm, out_hbm,
                 idx_v, rows_v, q_v, out_v, sem):
    wid = lax.axis_index("s") * 2 + lax.axis_index("c")

    def chunk(ci, carry):
        qbase = wid * _QPW + ci * _QCH
        ibase = qbase * K
        pltpu.sync_copy(idx_hbm.at[pl.ds(ibase, _ROWS)], idx_v)
        pltpu.sync_copy(qproj_hbm.at[pl.ds(qbase, _QCH)], q_v)
        pltpu.async_copy(table_hbm.at[idx_v], rows_v, sem).wait()

        def per_q(qi, c2):
            def per_c(cj, c3):
                qvec = q_v[qi, pl.ds(cj * 16, 16)]
                acc = jnp.full((16,), -jnp.inf, jnp.float32)
                for k in range(K):
                    x = rows_v[qi * K + k, pl.ds(cj * 16, 16)] + qvec
                    acc = jnp.maximum(acc, jnp.maximum(x, 0.2 * x))
                out_v[qi, pl.ds(cj * 16, 16)] = acc
                return c3
            return lax.fori_loop(0, _CV, per_c, c2)

        lax.fori_loop(0, _QCH, per_q, 0)
        pltpu.sync_copy(out_v, out_hbm.at[pl.ds(qbase, _QCH)])
        return carry

    lax.fori_loop(0, _NCH, chunk, 0)


def _gather_maxpool(table, idx_flat, qproj_flat):
    mesh = plsc.VectorSubcoreMesh(core_axis_name="c", subcore_axis_name="s",
                                  num_cores=2, num_subcores=16)
    fn = functools.partial(
        pl.kernel,
        out_type=jax.ShapeDtypeStruct((B * NQ, C), jnp.float32),
        mesh=mesh,
        scratch_types=[
            pltpu.VMEM((_ROWS,), jnp.int32),
            pltpu.VMEM((_ROWS, C), jnp.float32),
            pltpu.VMEM((_QCH, C), jnp.float32),
            pltpu.VMEM((_QCH, C), jnp.float32),
            pltpu.SemaphoreType.DMA,
        ],
    )(_gather_body)
    return fn(table, idx_flat, qproj_flat)


def kernel(query_coords, query_features, key_coords, key_features, W, b):
    w1 = W[:, :C]
    w2m1 = W[:, C:] - w1
    idx = _topk(query_coords, key_coords)
    kf_proj = _kproj(key_features, w1)
    q_proj = _qproj(query_features, w2m1, b.reshape(1, C))
    out = _gather_maxpool(kf_proj.reshape(B * NK, C),
                          idx.reshape(B * NQ * K),
                          q_proj.reshape(B * NQ, C))
    return out.reshape(B, NQ, C)


# speedup vs baseline: 2.6431x; 1.0461x over previous
import functools

import jax
import jax.numpy as jnp
from jax import lax
from jax.experimental import pallas as pl
from jax.experimental.pallas import tpu as pltpu
from jax.experimental.pallas import tpu_sc as plsc

B, NQ, NK, C, K = 2, 2048, 8192, 256, 16
TQ = 256
TN = 2048


def _topk_body(qt_ref, kc_ref, idx_ref):
    b = pl.program_id(0)
    q = qt_ref[0]
    kc = kc_ref[0]
    qb = q.astype(jnp.bfloat16).astype(jnp.float32)
    kb = kc.astype(jnp.bfloat16).astype(jnp.float32)
    q2 = jnp.zeros((TQ, 1), jnp.float32)
    k2 = jnp.zeros((1, NK), jnp.float32)
    dot = jnp.zeros((TQ, NK), jnp.float32)
    for d in range(3):
        q2 = q2 + q[:, d:d + 1] * q[:, d:d + 1]
        k2 = k2 + kc[d:d + 1, :] * kc[d:d + 1, :]
        dot = dot + qb[:, d:d + 1] * kb[d:d + 1, :]
    dist = (q2 + k2) - 2.0 * dot
    colrev = (NK - lax.broadcasted_iota(jnp.int32, (TQ, NK), 1)).astype(
        jnp.float32)
    cols = []
    for t in range(K):
        m = jnp.min(dist, axis=1, keepdims=True)
        ar = jnp.max(jnp.where(dist == m, colrev, 0.0), axis=1, keepdims=True)
        cols.append(ar)
        if t < K - 1:
            dist = jnp.where(colrev == ar, jnp.inf, dist)
    idx = (float(NK) - jnp.concatenate(cols, axis=1)).astype(jnp.int32)
    idx_ref[0] = idx + b * NK


def _topk(query_coords, key_coords):
    qt = jnp.transpose(query_coords, (0, 2, 1))
    return pl.pallas_call(
        _topk_body,
        grid=(B, NQ // TQ),
        in_specs=[
            pl.BlockSpec((1, TQ, 3), lambda b, i: (b, i, 0)),
            pl.BlockSpec((1, 3, NK), lambda b, i: (b, 0, 0)),
        ],
        out_specs=pl.BlockSpec((1, TQ, K), lambda b, i: (b, i, 0)),
        out_shape=jax.ShapeDtypeStruct((B, NQ, K), jnp.int32),
    )(qt, key_coords)


def _kproj_body(x_ref, w_ref, o_ref):
    o_ref[0] = lax.dot_general(x_ref[0], w_ref[...],
                               (((0,), (1,)), ((), ())),
                               preferred_element_type=jnp.float32)


def _qproj_body(x_ref, w_ref, b_ref, o_ref):
    o_ref[0] = lax.dot_general(x_ref[0], w_ref[...],
                               (((0,), (1,)), ((), ())),
                               preferred_element_type=jnp.float32) + b_ref[...]


def _kproj(key_features, w1):
    return pl.pallas_call(
        _kproj_body,
        grid=(B, NK // TN),
        in_specs=[
            pl.BlockSpec((1, C, TN), lambda b, i: (b, 0, i)),
            pl.BlockSpec((C, C), lambda b, i: (0, 0)),
        ],
        out_specs=pl.BlockSpec((1, TN, C), lambda b, i: (b, i, 0)),
        out_shape=jax.ShapeDtypeStruct((B, NK, C), jnp.float32),
    )(key_features, w1)


def _qproj(query_features, w2m1, bias):
    return pl.pallas_call(
        _qproj_body,
        grid=(B, NQ // TN),
        in_specs=[
            pl.BlockSpec((1, C, TN), lambda b, i: (b, 0, i)),
            pl.BlockSpec((C, C), lambda b, i: (0, 0)),
            pl.BlockSpec((1, C), lambda b, i: (0, 0)),
        ],
        out_specs=pl.BlockSpec((1, TN, C), lambda b, i: (b, i, 0)),
        out_shape=jax.ShapeDtypeStruct((B, NQ, C), jnp.float32),
    )(query_features, w2m1, bias)


_NW = 32
_QPW = (B * NQ) // _NW
_QCH = 8
_NCH = _QPW // _QCH
_ROWS = _QCH * K
_CV = C // 16


def _gather_body(table_hbm, idx_hbm, qproj_hbm, out_hbm,
                 idx_v, rows_v, q_v, out_v, sem):
    wid = lax.axis_index("s") * 2 + lax.axis_index("c")

    def chunk(ci, carry):
        qbase = wid * _QPW + ci * _QCH
        ibase = qbase * K
        pltpu.sync_copy(idx_hbm.at[pl.ds(ibase, _ROWS)], idx_v)
        pltpu.sync_copy(qproj_hbm.at[pl.ds(qbase, _QCH)], q_v)
        pltpu.async_copy(table_hbm.at[idx_v], rows_v, sem).wait()

        def per_q(qi, c2):
            def per_c(cj, c3):
                qvec = q_v[qi, pl.ds(cj * 16, 16)]
                acc = jnp.full((16,), -jnp.inf, jnp.float32)
                for k in range(K):
                    x = rows_v[qi * K + k, pl.ds(cj * 16, 16)] + qvec
                    acc = jnp.maximum(acc, jnp.maximum(x, 0.2 * x))
                out_v[qi, pl.ds(cj * 16, 16)] = acc
                return c3
            return lax.fori_loop(0, _CV, per_c, c2)

        lax.fori_loop(0, _QCH, per_q, 0)
        pltpu.sync_copy(out_v, out_hbm.at[pl.ds(qbase, _QCH)])
        return carry

    lax.fori_loop(0, _NCH, chunk, 0)


def _gather_maxpool(table, idx_flat, qproj_flat):
    mesh = plsc.VectorSubcoreMesh(core_axis_name="c", subcore_axis_name="s",
                                  num_cores=2, num_subcores=16)
    fn = functools.partial(
        pl.kernel,
        out_type=jax.ShapeDtypeStruct((B * NQ, C), jnp.float32),
        mesh=mesh,
        scratch_types=[
            pltpu.VMEM((_ROWS,), jnp.int32),
            pltpu.VMEM((_ROWS, C), jnp.float32),
            pltpu.VMEM((_QCH, C), jnp.float32),
            pltpu.VMEM((_QCH, C), jnp.float32),
            pltpu.SemaphoreType.DMA,
        ],
    )(_gather_body)
    return fn(table, idx_flat, qproj_flat)


def kernel(query_coords, query_features, key_coords, key_features, W, b):
    w1 = W[:, :C]
    w2m1 = W[:, C:] - w1
    idx = _topk(query_coords, key_coords)
    kf_proj = _kproj(key_features, w1)
    q_proj = _qproj(query_features, w2m1, b.reshape(1, C))
    out = _gather_maxpool(kf_proj.reshape(B * NK, C),
                          idx.reshape(B * NQ * K),
                          q_proj.reshape(B * NQ, C))
    return out.reshape(B, NQ, C)


# speedup vs baseline: 2.6885x; 1.0172x over previous
import functools

import jax
import jax.numpy as jnp
from jax import lax
from jax.experimental import pallas as pl
from jax.experimental.pallas import tpu as pltpu
from jax.experimental.pallas import tpu_sc as plsc

B, NQ, NK, C, K = 2, 2048, 8192, 256, 16
TQ = 256
TN = 2048


def _topk_body(boffset, qt_ref, kc_ref, idx_ref):
    q = qt_ref[...]
    kc = kc_ref[...]
    qb = q.astype(jnp.bfloat16).astype(jnp.float32)
    kb = kc.astype(jnp.bfloat16).astype(jnp.float32)
    q2 = jnp.zeros((TQ, 1), jnp.float32)
    k2 = jnp.zeros((1, NK), jnp.float32)
    dot = jnp.zeros((TQ, NK), jnp.float32)
    for d in range(3):
        q2 = q2 + q[:, d:d + 1] * q[:, d:d + 1]
        k2 = k2 + kc[d:d + 1, :] * kc[d:d + 1, :]
        dot = dot + qb[:, d:d + 1] * kb[d:d + 1, :]
    dist = (q2 + k2) - 2.0 * dot
    colrev = (NK - lax.broadcasted_iota(jnp.int32, (TQ, NK), 1)).astype(
        jnp.float32)
    cols = []
    for t in range(K):
        m = jnp.min(dist, axis=1, keepdims=True)
        ar = jnp.max(jnp.where(dist == m, colrev, 0.0), axis=1, keepdims=True)
        cols.append(ar)
        if t < K - 1:
            dist = jnp.where(colrev == ar, jnp.inf, dist)
    idx = (float(NK) - jnp.concatenate(cols, axis=1)).astype(jnp.int32)
    idx_ref[...] = idx + boffset


def _topk_1b(qt_b, kc_b, boffset):
    return pl.pallas_call(
        functools.partial(_topk_body, boffset),
        grid=(NQ // TQ,),
        in_specs=[
            pl.BlockSpec((TQ, 3), lambda i: (i, 0)),
            pl.BlockSpec((3, NK), lambda i: (0, 0)),
        ],
        out_specs=pl.BlockSpec((TQ, K), lambda i: (i, 0)),
        out_shape=jax.ShapeDtypeStruct((NQ, K), jnp.int32),
    )(qt_b, kc_b)


def _kproj_body(x_ref, w_ref, o_ref):
    o_ref[0] = lax.dot_general(x_ref[0], w_ref[...],
                               (((0,), (1,)), ((), ())),
                               preferred_element_type=jnp.float32)


def _qproj_body(x_ref, w_ref, b_ref, o_ref):
    o_ref[0] = lax.dot_general(x_ref[0], w_ref[...],
                               (((0,), (1,)), ((), ())),
                               preferred_element_type=jnp.float32) + b_ref[...]


def _kproj(key_features, w1):
    return pl.pallas_call(
        _kproj_body,
        grid=(B, NK // TN),
        in_specs=[
            pl.BlockSpec((1, C, TN), lambda b, i: (b, 0, i)),
            pl.BlockSpec((C, C), lambda b, i: (0, 0)),
        ],
        out_specs=pl.BlockSpec((1, TN, C), lambda b, i: (b, i, 0)),
        out_shape=jax.ShapeDtypeStruct((B, NK, C), jnp.float32),
    )(key_features, w1)


def _qproj(query_features, w2m1, bias):
    return pl.pallas_call(
        _qproj_body,
        grid=(B, NQ // TN),
        in_specs=[
            pl.BlockSpec((1, C, TN), lambda b, i: (b, 0, i)),
            pl.BlockSpec((C, C), lambda b, i: (0, 0)),
            pl.BlockSpec((1, C), lambda b, i: (0, 0)),
        ],
        out_specs=pl.BlockSpec((1, TN, C), lambda b, i: (b, i, 0)),
        out_shape=jax.ShapeDtypeStruct((B, NQ, C), jnp.float32),
    )(query_features, w2m1, bias)


_NW = 32
_QPW = NQ // _NW
_QCH = 8
_NCH = _QPW // _QCH
_ROWS = _QCH * K
_CV = C // 16


def _gather_body(table_hbm, idx_hbm, qproj_hbm, out_hbm,
                 idx_v, rows_v, q_v, out_v, sem):
    wid = lax.axis_index("s") * 2 + lax.axis_index("c")

    def chunk(ci, carry):
        qbase = wid * _QPW + ci * _QCH
        ibase = qbase * K
        pltpu.sync_copy(idx_hbm.at[pl.ds(ibase, _ROWS)], idx_v)
        pltpu.sync_copy(qproj_hbm.at[pl.ds(qbase, _QCH)], q_v)
        pltpu.async_copy(table_hbm.at[idx_v], rows_v, sem).wait()

        def per_q(qi, c2):
            def per_c(cj, c3):
                qvec = q_v[qi, pl.ds(cj * 16, 16)]
                acc = jnp.full((16,), -jnp.inf, jnp.float32)
                for k in range(K):
                    x = rows_v[qi * K + k, pl.ds(cj * 16, 16)] + qvec
                    acc = jnp.maximum(acc, jnp.maximum(x, 0.2 * x))
                out_v[qi, pl.ds(cj * 16, 16)] = acc
                return c3
            return lax.fori_loop(0, _CV, per_c, c2)

        lax.fori_loop(0, _QCH, per_q, 0)
        pltpu.sync_copy(out_v, out_hbm.at[pl.ds(qbase, _QCH)])
        return carry

    lax.fori_loop(0, _NCH, chunk, 0)


def _gather_maxpool(table, idx_flat, qproj_flat):
    mesh = plsc.VectorSubcoreMesh(core_axis_name="c", subcore_axis_name="s",
                                  num_cores=2, num_subcores=16)
    fn = functools.partial(
        pl.kernel,
        out_type=jax.ShapeDtypeStruct((NQ, C), jnp.float32),
        mesh=mesh,
        scratch_types=[
            pltpu.VMEM((_ROWS,), jnp.int32),
            pltpu.VMEM((_ROWS, C), jnp.float32),
            pltpu.VMEM((_QCH, C), jnp.float32),
            pltpu.VMEM((_QCH, C), jnp.float32),
            pltpu.SemaphoreType.DMA,
        ],
    )(_gather_body)
    return fn(table, idx_flat, qproj_flat)


def kernel(query_coords, query_features, key_coords, key_features, W, b):
    w1 = W[:, :C]
    w2m1 = W[:, C:] - w1
    kf_proj = _kproj(key_features, w1)
    q_proj = _qproj(query_features, w2m1, b.reshape(1, C))
    table = kf_proj.reshape(B * NK, C)
    qt = jnp.transpose(query_coords, (0, 2, 1))
    outs = []
    for bi in range(B):
        idx_b = _topk_1b(qt[bi], key_coords[bi], bi * NK)
        outs.append(_gather_maxpool(table, idx_b.reshape(NQ * K), q_proj[bi]))
    return jnp.stack(outs, axis=0)


# speedup vs baseline: 2.8623x; 1.0647x over previous
import functools

import jax
import jax.numpy as jnp
from jax import lax
from jax.experimental import pallas as pl
from jax.experimental.pallas import tpu as pltpu
from jax.experimental.pallas import tpu_sc as plsc

B, NQ, NK, C, K = 2, 2048, 8192, 256, 16
TQ = 256
TN = 2048


def _topk_body(boffset, qt_ref, kc_ref, idx_ref):
    q = qt_ref[...]
    kc = kc_ref[...]
    qb = q.astype(jnp.bfloat16)
    kb = kc.astype(jnp.bfloat16)
    dot = lax.dot_general(qb, kb, (((1,), (0,)), ((), ())),
                          preferred_element_type=jnp.float32)
    q2 = jnp.zeros((TQ, 1), jnp.float32)
    k2 = jnp.zeros((1, NK), jnp.float32)
    for d in range(3):
        q2 = q2 + q[:, d:d + 1] * q[:, d:d + 1]
        k2 = k2 + kc[d:d + 1, :] * kc[d:d + 1, :]
    dist = (q2 + k2) - 2.0 * dot
    colrev = (NK - lax.broadcasted_iota(jnp.int32, (TQ, NK), 1)).astype(
        jnp.float32)
    cols = []
    for t in range(K):
        m = jnp.min(dist, axis=1, keepdims=True)
        ar = jnp.max(jnp.where(dist == m, colrev, 0.0), axis=1, keepdims=True)
        cols.append(ar)
        if t < K - 1:
            dist = jnp.where(colrev == ar, jnp.inf, dist)
    idx = (float(NK) - jnp.concatenate(cols, axis=1)).astype(jnp.int32)
    idx_ref[...] = idx + boffset


def _topk_1b(qt_b, kc_b, boffset):
    return pl.pallas_call(
        functools.partial(_topk_body, boffset),
        grid=(NQ // TQ,),
        in_specs=[
            pl.BlockSpec((TQ, 3), lambda i: (i, 0)),
            pl.BlockSpec((3, NK), lambda i: (0, 0)),
        ],
        out_specs=pl.BlockSpec((TQ, K), lambda i: (i, 0)),
        out_shape=jax.ShapeDtypeStruct((NQ, K), jnp.int32),
    )(qt_b, kc_b)


def _kproj_body(x_ref, w_ref, o_ref):
    o_ref[0] = lax.dot_general(x_ref[0], w_ref[...],
                               (((0,), (1,)), ((), ())),
                               preferred_element_type=jnp.float32)


def _qproj_body(x_ref, w_ref, b_ref, o_ref):
    o_ref[0] = lax.dot_general(x_ref[0], w_ref[...],
                               (((0,), (1,)), ((), ())),
                               preferred_element_type=jnp.float32) + b_ref[...]


def _kproj(key_features, w1):
    return pl.pallas_call(
        _kproj_body,
        grid=(B, NK // TN),
        in_specs=[
            pl.BlockSpec((1, C, TN), lambda b, i: (b, 0, i)),
            pl.BlockSpec((C, C), lambda b, i: (0, 0)),
        ],
        out_specs=pl.BlockSpec((1, TN, C), lambda b, i: (b, i, 0)),
        out_shape=jax.ShapeDtypeStruct((B, NK, C), jnp.float32),
    )(key_features, w1)


def _qproj(query_features, w2m1, bias):
    return pl.pallas_call(
        _qproj_body,
        grid=(B, NQ // TN),
        in_specs=[
            pl.BlockSpec((1, C, TN), lambda b, i: (b, 0, i)),
            pl.BlockSpec((C, C), lambda b, i: (0, 0)),
            pl.BlockSpec((1, C), lambda b, i: (0, 0)),
        ],
        out_specs=pl.BlockSpec((1, TN, C), lambda b, i: (b, i, 0)),
        out_shape=jax.ShapeDtypeStruct((B, NQ, C), jnp.float32),
    )(query_features, w2m1, bias)


_NW = 32
_QPW = NQ // _NW
_QCH = 8
_NCH = _QPW // _QCH
_ROWS = _QCH * K
_CV = C // 16


def _gather_body(table_hbm, idx_hbm, qproj_hbm, out_hbm,
                 idx_v, rows_v, q_v, out_v, sem):
    wid = lax.axis_index("s") * 2 + lax.axis_index("c")

    def chunk(ci, carry):
        qbase = wid * _QPW + ci * _QCH
        ibase = qbase * K
        pltpu.sync_copy(idx_hbm.at[pl.ds(ibase, _ROWS)], idx_v)
        pltpu.sync_copy(qproj_hbm.at[pl.ds(qbase, _QCH)], q_v)
        pltpu.async_copy(table_hbm.at[idx_v], rows_v, sem).wait()

        def per_q(qi, c2):
            def per_c(cj, c3):
                qvec = q_v[qi, pl.ds(cj * 16, 16)]
                acc = jnp.full((16,), -jnp.inf, jnp.float32)
                for k in range(K):
                    x = rows_v[qi * K + k, pl.ds(cj * 16, 16)] + qvec
                    acc = jnp.maximum(acc, jnp.maximum(x, 0.2 * x))
                out_v[qi, pl.ds(cj * 16, 16)] = acc
                return c3
            return lax.fori_loop(0, _CV, per_c, c2)

        lax.fori_loop(0, _QCH, per_q, 0)
        pltpu.sync_copy(out_v, out_hbm.at[pl.ds(qbase, _QCH)])
        return carry

    lax.fori_loop(0, _NCH, chunk, 0)


def _gather_maxpool(table, idx_flat, qproj_flat):
    mesh = plsc.VectorSubcoreMesh(core_axis_name="c", subcore_axis_name="s",
                                  num_cores=2, num_subcores=16)
    fn = functools.partial(
        pl.kernel,
        out_type=jax.ShapeDtypeStruct((NQ, C), jnp.float32),
        mesh=mesh,
        scratch_types=[
            pltpu.VMEM((_ROWS,), jnp.int32),
            pltpu.VMEM((_ROWS, C), jnp.float32),
            pltpu.VMEM((_QCH, C), jnp.float32),
            pltpu.VMEM((_QCH, C), jnp.float32),
            pltpu.SemaphoreType.DMA,
        ],
    )(_gather_body)
    return fn(table, idx_flat, qproj_flat)


def kernel(query_coords, query_features, key_coords, key_features, W, b):
    w1 = W[:, :C]
    w2m1 = W[:, C:] - w1
    kf_proj = _kproj(key_features, w1)
    q_proj = _qproj(query_features, w2m1, b.reshape(1, C))
    table = kf_proj.reshape(B * NK, C)
    qt = jnp.transpose(query_coords, (0, 2, 1))
    outs = []
    for bi in range(B):
        idx_b = _topk_1b(qt[bi], key_coords[bi], bi * NK)
        outs.append(_gather_maxpool(table, idx_b.reshape(NQ * K), q_proj[bi]))
    return jnp.stack(outs, axis=0)


# speedup vs baseline: 2.9379x; 1.0264x over previous
import functools

import jax
import jax.numpy as jnp
from jax import lax
from jax.experimental import pallas as pl
from jax.experimental.pallas import tpu as pltpu
from jax.experimental.pallas import tpu_sc as plsc

B, NQ, NK, C, K = 2, 2048, 8192, 256, 16
TQ = 256
TN = 2048


def _topk_body(boffset, qt_ref, kc_ref, idx_ref):
    q = qt_ref[...]
    kc = kc_ref[...]
    qb = q.astype(jnp.bfloat16)
    kb = kc.astype(jnp.bfloat16)
    dot = lax.dot_general(qb, kb, (((1,), (0,)), ((), ())),
                          preferred_element_type=jnp.float32)
    q2 = jnp.zeros((TQ, 1), jnp.float32)
    k2 = jnp.zeros((1, NK), jnp.float32)
    for d in range(3):
        q2 = q2 + q[:, d:d + 1] * q[:, d:d + 1]
        k2 = k2 + kc[d:d + 1, :] * kc[d:d + 1, :]
    dist = (q2 + k2) - 2.0 * dot
    colrev = (NK - lax.broadcasted_iota(jnp.int32, (TQ, NK), 1)).astype(
        jnp.float32)
    cols = []
    for t in range(K):
        m = jnp.min(dist, axis=1, keepdims=True)
        ar = jnp.max(jnp.where(dist == m, colrev, 0.0), axis=1, keepdims=True)
        cols.append(ar)
        if t < K - 1:
            dist = jnp.where(colrev == ar, jnp.inf, dist)
    idx = (float(NK) - jnp.concatenate(cols, axis=1)).astype(jnp.int32)
    idx_ref[...] = idx + boffset


def _topk_1b(qt_b, kc_b, boffset):
    return pl.pallas_call(
        functools.partial(_topk_body, boffset),
        grid=(NQ // TQ,),
        in_specs=[
            pl.BlockSpec((TQ, 3), lambda i: (i, 0)),
            pl.BlockSpec((3, NK), lambda i: (0, 0)),
        ],
        out_specs=pl.BlockSpec((TQ, K), lambda i: (i, 0)),
        out_shape=jax.ShapeDtypeStruct((NQ, K), jnp.int32),
    )(qt_b, kc_b)


def _kproj_body(x_ref, w_ref, o_ref):
    o_ref[0] = lax.dot_general(x_ref[0], w_ref[...],
                               (((0,), (1,)), ((), ())),
                               preferred_element_type=jnp.float32)


def _qproj_body(x_ref, w_ref, b_ref, o_ref):
    o_ref[0] = lax.dot_general(x_ref[0], w_ref[...],
                               (((0,), (1,)), ((), ())),
                               preferred_element_type=jnp.float32) + b_ref[...]


def _kproj(key_features, w1):
    return pl.pallas_call(
        _kproj_body,
        grid=(B, NK // TN),
        in_specs=[
            pl.BlockSpec((1, C, TN), lambda b, i: (b, 0, i)),
            pl.BlockSpec((C, C), lambda b, i: (0, 0)),
        ],
        out_specs=pl.BlockSpec((1, TN, C), lambda b, i: (b, i, 0)),
        out_shape=jax.ShapeDtypeStruct((B, NK, C), jnp.float32),
    )(key_features, w1)


def _qproj(query_features, w2m1, bias):
    return pl.pallas_call(
        _qproj_body,
        grid=(B, NQ // TN),
        in_specs=[
            pl.BlockSpec((1, C, TN), lambda b, i: (b, 0, i)),
            pl.BlockSpec((C, C), lambda b, i: (0, 0)),
            pl.BlockSpec((1, C), lambda b, i: (0, 0)),
        ],
        out_specs=pl.BlockSpec((1, TN, C), lambda b, i: (b, i, 0)),
        out_shape=jax.ShapeDtypeStruct((B, NQ, C), jnp.float32),
    )(query_features, w2m1, bias)


_NW = 32
_QPW = NQ // _NW
_QCH = 8
_NCH = _QPW // _QCH
_ROWS = _QCH * K
_CV = C // 16


def _gather_body(table_hbm, idx_hbm, qproj_hbm, out_hbm,
                 idx_v, rows_v, q_v, out_v, sem0, sem1):
    wid = lax.axis_index("s") * 2 + lax.axis_index("c")
    sems = (sem0, sem1)

    def stage(ci, p):
        qbase = wid * _QPW + ci * _QCH
        pltpu.sync_copy(idx_hbm.at[pl.ds(qbase * K, _ROWS)], idx_v.at[p])
        pltpu.sync_copy(qproj_hbm.at[pl.ds(qbase, _QCH)], q_v.at[p])
        return pltpu.async_copy(table_hbm.at[idx_v.at[p]], rows_v.at[p],
                                sems[p])

    pending = stage(0, 0)
    for ci in range(_NCH):
        p = ci & 1
        nxt = stage(ci + 1, p ^ 1) if ci + 1 < _NCH else None
        pending.wait()

        def per_q(qi, c2):
            def per_c(cj, c3):
                qvec = q_v[p, qi, pl.ds(cj * 16, 16)]
                acc = jnp.full((16,), -jnp.inf, jnp.float32)
                for k in range(K):
                    x = rows_v[p, qi * K + k, pl.ds(cj * 16, 16)] + qvec
                    acc = jnp.maximum(acc, jnp.maximum(x, 0.2 * x))
                out_v[p, qi, pl.ds(cj * 16, 16)] = acc
                return c3
            return lax.fori_loop(0, _CV, per_c, c2)

        lax.fori_loop(0, _QCH, per_q, 0)
        qbase = wid * _QPW + ci * _QCH
        pltpu.sync_copy(out_v.at[p], out_hbm.at[pl.ds(qbase, _QCH)])
        pending = nxt


def _gather_maxpool(table, idx_flat, qproj_flat):
    mesh = plsc.VectorSubcoreMesh(core_axis_name="c", subcore_axis_name="s",
                                  num_cores=2, num_subcores=16)
    fn = functools.partial(
        pl.kernel,
        out_type=jax.ShapeDtypeStruct((NQ, C), jnp.float32),
        mesh=mesh,
        scratch_types=[
            pltpu.VMEM((2, _ROWS), jnp.int32),
            pltpu.VMEM((2, _ROWS, C), jnp.float32),
            pltpu.VMEM((2, _QCH, C), jnp.float32),
            pltpu.VMEM((2, _QCH, C), jnp.float32),
            pltpu.SemaphoreType.DMA,
            pltpu.SemaphoreType.DMA,
        ],
    )(_gather_body)
    return fn(table, idx_flat, qproj_flat)


def kernel(query_coords, query_features, key_coords, key_features, W, b):
    w1 = W[:, :C]
    w2m1 = W[:, C:] - w1
    kf_proj = _kproj(key_features, w1)
    q_proj = _qproj(query_features, w2m1, b.reshape(1, C))
    table = kf_proj.reshape(B * NK, C)
    qt = jnp.transpose(query_coords, (0, 2, 1))
    outs = []
    for bi in range(B):
        idx_b = _topk_1b(qt[bi], key_coords[bi], bi * NK)
        outs.append(_gather_maxpool(table, idx_b.reshape(NQ * K), q_proj[bi]))
    return jnp.stack(outs, axis=0)
